# baseline (speedup 1.0000x reference)
import functools

import jax
import jax.numpy as jnp
from jax import lax
from jax.experimental import pallas as pl
from jax.experimental.pallas import tpu as pltpu
from jax.experimental.pallas import tpu_sc as plsc

S, R, C = 38, 340, 24
NUM_LAYERS = 4
EPS = 1e-5
B = 256
N = B * (S + R)
E = N * 16
NS = 16
NC = 2
RPT = N // NS
EPT = E // NS
EPW = E // (NS * NC)
KA = 864
KD = 1512
RB = 2016


def _leaky(v):
    return jnp.where(v >= 0, v, 0.2 * v)


RCH = 864


def _make_deg_body(ch):
    def _deg_body(dst_hbm, zeros_hbm, ones_hbm, out_hbm, dstb, onesb, zbuf, acc):
        s = lax.axis_index("s")
        pltpu.sync_copy(zeros_hbm, zbuf)
        pltpu.sync_copy(ones_hbm, onesb)
        for j in range(RPT // RCH):
            pltpu.sync_copy(zbuf, acc.at[pl.ds(s * RPT + j * RCH, RCH)])
        plsc.subcore_barrier()
        base = (ch * NS + s) * EPW

        def chunk(i, carry):
            off = base + i * KD
            pltpu.sync_copy(dst_hbm.at[pl.ds(off, KD)], dstb)
            pltpu.sync_copy(onesb, acc.at[dstb], add=True)
            return carry

        lax.fori_loop(0, EPW // KD, chunk, 0)
        plsc.subcore_barrier()
        for j in range(RPT // RCH):
            pltpu.sync_copy(acc.at[pl.ds(s * RPT + j * RCH, RCH)], zbuf)
            pltpu.sync_copy(zbuf, out_hbm.at[pl.ds(s * RPT + j * RCH, RCH)])
    return _deg_body


def _make_agg_body(ch):
    def _agg_body(src_hbm, dst_hbm, h2_hbm, out_hbm,
                  srcA, dstA, rowsA, srcB, dstB, rowsB, semA, semB, acc):
        s = lax.axis_index("s")
        for j in range(RPT // RCH):
            pltpu.sync_copy(h2_hbm.at[pl.ds(ch * N + s * RPT + j * RCH, RCH)],
                            rowsA.at[pl.ds(0, RCH)])
            pltpu.sync_copy(rowsA.at[pl.ds(0, RCH)],
                            acc.at[pl.ds(s * RPT + j * RCH, RCH)])
        plsc.subcore_barrier()
        base = s * EPT
        npair = EPT // KA // 2

        pltpu.sync_copy(src_hbm.at[pl.ds(base, KA)], srcA)
        pltpu.sync_copy(dst_hbm.at[pl.ds(base, KA)], dstA)
        pltpu.async_copy(h2_hbm.at[srcA], rowsA, semA)

        def pair(j, carry):
            offB = base + (2 * j + 1) * KA
            pltpu.sync_copy(src_hbm.at[pl.ds(offB, KA)], srcB)
            pltpu.sync_copy(dst_hbm.at[pl.ds(offB, KA)], dstB)
            pltpu.async_copy(h2_hbm.at[srcB], rowsB, semB)
            pltpu.make_async_copy(h2_hbm.at[srcA], rowsA, semA).wait()
            pltpu.sync_copy(rowsA, acc.at[dstA], add=True)

            @pl.when(j < npair - 1)
            def _():
                offA = base + (2 * j + 2) * KA
                pltpu.sync_copy(src_hbm.at[pl.ds(offA, KA)], srcA)
                pltpu.sync_copy(dst_hbm.at[pl.ds(offA, KA)], dstA)
                pltpu.async_copy(h2_hbm.at[srcA], rowsA, semA)

            pltpu.make_async_copy(h2_hbm.at[srcB], rowsB, semB).wait()
            pltpu.sync_copy(rowsB, acc.at[dstB], add=True)
            return carry

        lax.fori_loop(0, npair, pair, 0)
        plsc.subcore_barrier()
        for j in range(RPT // RCH):
            pltpu.sync_copy(acc.at[pl.ds(s * RPT + j * RCH, RCH)],
                            rowsA.at[pl.ds(0, RCH)])
            pltpu.sync_copy(rowsA.at[pl.ds(0, RCH)],
                            out_hbm.at[pl.ds(s * RPT + j * RCH, RCH)])
    return _agg_body


def _sc_mesh():
    return plsc.VectorSubcoreMesh(core_axis_name="c", subcore_axis_name="s",
                                  num_cores=1)


def _deg_call(dst, ch):
    fn = pl.kernel(
        _make_deg_body(ch),
        out_type=jax.ShapeDtypeStruct((N,), jnp.float32),
        mesh=_sc_mesh(),
        scratch_types=[
            pltpu.VMEM((KD,), jnp.int32),
            pltpu.VMEM((KD,), jnp.float32),
            pltpu.VMEM((RCH,), jnp.float32),
            pltpu.VMEM_SHARED((N,), jnp.float32),
        ],
        compiler_params=pltpu.CompilerParams(use_tc_tiling_on_sc=False),
    )
    return fn(dst, jnp.zeros((RCH,), jnp.float32), jnp.ones((KD,), jnp.float32))


def _agg_call(src_h, dst, h2, ch):
    fn = pl.kernel(
        _make_agg_body(ch),
        out_type=jax.ShapeDtypeStruct((N, 16), jnp.float32),
        mesh=_sc_mesh(),
        scratch_types=[
            pltpu.VMEM((KA,), jnp.int32),
            pltpu.VMEM((KA,), jnp.int32),
            pltpu.VMEM((KA, 16), jnp.float32),
            pltpu.VMEM((KA,), jnp.int32),
            pltpu.VMEM((KA,), jnp.int32),
            pltpu.VMEM((KA, 16), jnp.float32),
            pltpu.SemaphoreType.DMA,
            pltpu.SemaphoreType.DMA,
            pltpu.VMEM_SHARED((N, 16), jnp.float32),
        ],
        compiler_params=pltpu.CompilerParams(use_tc_tiling_on_sc=False),
    )
    return fn(src_h, dst, h2)


PG = 16
SBK = B * S // PG
RBK = B * R // PG


def _stats_body(sx_ref, rx_ref, ss_ref, rs_ref):
    i = pl.program_id(0)
    sx = sx_ref[...]
    rflat = rx_ref[...][:, :3]

    @pl.when(i == 0)
    def _():
        ss_ref[...] = jnp.zeros_like(ss_ref)
        rs_ref[...] = jnp.zeros_like(rs_ref)

    ss_ref[0, :] += jnp.sum(sx, axis=0)
    ss_ref[1, :] += jnp.sum(sx * sx, axis=0)
    rs_ref[0, :] += jnp.sum(rflat, axis=0)
    rs_ref[1, :] += jnp.sum(rflat * rflat, axis=0)


def _stats_call(s_x, r_x):
    return pl.pallas_call(
        _stats_body,
        grid=(PG,),
        in_specs=[
            pl.BlockSpec((SBK, 4), lambda i: (i, 0)),
            pl.BlockSpec((RBK, 8), lambda i: (i, 0)),
        ],
        out_specs=[
            pl.BlockSpec((2, 4), lambda i: (0, 0)),
            pl.BlockSpec((2, 3), lambda i: (0, 0)),
        ],
        out_shape=[
            jax.ShapeDtypeStruct((2, 4), jnp.float32),
            jax.ShapeDtypeStruct((2, 3), jnp.float32),
        ],
    )(s_x, r_x)


def _prologue_body(sx_ref, rx_ref, ss_ref, rs_ref, bnsw, bnsb, bnrw, bnrb,
                   lsw, lrw, s_out, r_out, ox_out):
    ns = float(B * S)
    mu = ss_ref[0, :] / ns
    var = ss_ref[1, :] / ns - mu * mu
    sn = (sx_ref[...] - mu) / jnp.sqrt(var + EPS) * bnsw[0] + bnsb[0]
    s_out[...] = _leaky(jnp.dot(sn, lsw[...].T, preferred_element_type=jnp.float32))

    rx = rx_ref[...]
    nr = float(B * R)
    mu_r = rs_ref[0, :] / nr
    var_r = rs_ref[1, :] / nr - mu_r * mu_r
    rn = (rx[:, :3] - mu_r) / jnp.sqrt(var_r + EPS) * bnrw[0] + bnrb[0]
    r_out[...] = _leaky(jnp.dot(rn, lrw[...].T, preferred_element_type=jnp.float32))

    ox_out[...] = rx.reshape(RBK // R, R, 8)[:, 0, 3:8]


def _prologue_call(s_x, r_x, bn_s_w, bn_s_b, bn_r_w, bn_r_b, lin_s_W, lin_r_W):
    sstats, rstats = _stats_call(s_x, r_x)
    return pl.pallas_call(
        _prologue_body,
        grid=(PG,),
        in_specs=[
            pl.BlockSpec((SBK, 4), lambda i: (i, 0)),
            pl.BlockSpec((RBK, 8), lambda i: (i, 0)),
            pl.BlockSpec((2, 4), lambda i: (0, 0)),
            pl.BlockSpec((2, 3), lambda i: (0, 0)),
            pl.BlockSpec((1, 4), lambda i: (0, 0)),
            pl.BlockSpec((1, 4), lambda i: (0, 0)),
            pl.BlockSpec((1, 3), lambda i: (0, 0)),
            pl.BlockSpec((1, 3), lambda i: (0, 0)),
            pl.BlockSpec((C, 4), lambda i: (0, 0)),
            pl.BlockSpec((C, 3), lambda i: (0, 0)),
        ],
        out_specs=[
            pl.BlockSpec((SBK, C), lambda i: (i, 0)),
            pl.BlockSpec((RBK, C), lambda i: (i, 0)),
            pl.BlockSpec((RBK // R, 5), lambda i: (i, 0)),
        ],
        out_shape=[
            jax.ShapeDtypeStruct((B * S, C), jnp.float32),
            jax.ShapeDtypeStruct((B * R, C), jnp.float32),
            jax.ShapeDtypeStruct((B, 5), jnp.float32),
        ],
    )(s_x, r_x, sstats, rstats, bn_s_w.reshape(1, 4), bn_s_b.reshape(1, 4),
      bn_r_w.reshape(1, 3), bn_r_b.reshape(1, 3), lin_s_W, lin_r_W)


def _split_h(h, h2_ref):
    rb = h.shape[0]
    h2_ref[0] = h[:, :16]
    h2_ref[1] = jnp.concatenate([h[:, 16:24], jnp.zeros((rb, 8), jnp.float32)], axis=1)


def _dense0_body(x_ref, deg0_ref, deg1_ref, w_ref, dinv_ref, h2_ref):
    deg = 1.0 + deg0_ref[:, 0] + deg1_ref[:, 0]
    dinv = lax.rsqrt(deg)
    dinv_ref[:, 0] = dinv
    h = jnp.dot(x_ref[...], w_ref[...].T, preferred_element_type=jnp.float32)
    _split_h(h * dinv[:, None], h2_ref)


def _dense0_call(x0, deg0, deg1, W0):
    grid = (N // RB,)
    return pl.pallas_call(
        _dense0_body,
        grid=grid,
        in_specs=[
            pl.BlockSpec((RB, C), lambda i: (i, 0)),
            pl.BlockSpec((RB, 1), lambda i: (i, 0)),
            pl.BlockSpec((RB, 1), lambda i: (i, 0)),
            pl.BlockSpec((C, C), lambda i: (0, 0)),
        ],
        out_specs=[
            pl.BlockSpec((RB, 1), lambda i: (i, 0)),
            pl.BlockSpec((2, RB, 16), lambda i: (0, i, 0)),
        ],
        out_shape=[
            jax.ShapeDtypeStruct((N, 1), jnp.float32),
            jax.ShapeDtypeStruct((2, N, 16), jnp.float32),
        ],
    )(x0, deg0.reshape(N, 1), deg1.reshape(N, 1), W0)


def _dense_body(x_ref, agg0_ref, agg1_ref, dinv_ref, b_ref, w_ref, xn_ref, h2_ref):
    dinv = dinv_ref[:, 0]
    agg = jnp.concatenate([agg0_ref[...], agg1_ref[:, :8]], axis=1)
    xn = x_ref[...] + _leaky(dinv[:, None] * agg + b_ref[0])
    xn_ref[...] = xn
    h = jnp.dot(xn, w_ref[...].T, preferred_element_type=jnp.float32)
    _split_h(h * dinv[:, None], h2_ref)


def _dense_call(x, agg0, agg1, dinv, b, Wnext):
    grid = (N // RB,)
    return pl.pallas_call(
        _dense_body,
        grid=grid,
        in_specs=[
            pl.BlockSpec((RB, C), lambda i: (i, 0)),
            pl.BlockSpec((RB, 16), lambda i: (i, 0)),
            pl.BlockSpec((RB, 16), lambda i: (i, 0)),
            pl.BlockSpec((RB, 1), lambda i: (i, 0)),
            pl.BlockSpec((1, C), lambda i: (0, 0)),
            pl.BlockSpec((C, C), lambda i: (0, 0)),
        ],
        out_specs=[
            pl.BlockSpec((RB, C), lambda i: (i, 0)),
            pl.BlockSpec((2, RB, 16), lambda i: (0, i, 0)),
        ],
        out_shape=[
            jax.ShapeDtypeStruct((N, C), jnp.float32),
            jax.ShapeDtypeStruct((2, N, 16), jnp.float32),
        ],
    )(x, agg0, agg1, dinv, b.reshape(1, C), Wnext)


def _dense_last_body(x_ref, agg0_ref, agg1_ref, dinv_ref, b_ref, xn_ref):
    dinv = dinv_ref[:, 0]
    agg = jnp.concatenate([agg0_ref[...], agg1_ref[:, :8]], axis=1)
    xn_ref[...] = x_ref[...] + _leaky(dinv[:, None] * agg + b_ref[0])


def _dense_last_call(x, agg0, agg1, dinv, b):
    grid = (N // RB,)
    return pl.pallas_call(
        _dense_last_body,
        grid=grid,
        in_specs=[
            pl.BlockSpec((RB, C), lambda i: (i, 0)),
            pl.BlockSpec((RB, 16), lambda i: (i, 0)),
            pl.BlockSpec((RB, 16), lambda i: (i, 0)),
            pl.BlockSpec((RB, 1), lambda i: (i, 0)),
            pl.BlockSpec((1, C), lambda i: (0, 0)),
        ],
        out_specs=pl.BlockSpec((RB, C), lambda i: (i, 0)),
        out_shape=jax.ShapeDtypeStruct((N, C), jnp.float32),
    )(x, agg0, agg1, dinv, b.reshape(1, C))


def _readout_body(x_ref, ox_ref, lrw, lrb, w1, b1, w2, b2, out_ref):
    xg = jnp.mean(x_ref[...], axis=2)
    logits = jnp.dot(xg, lrw[...].T, preferred_element_type=jnp.float32) + lrb[0]
    exl = jnp.exp(logits)
    p = exl / (jnp.sum(exl, axis=1, keepdims=True) + 1.0)
    o = _leaky(jnp.dot(ox_ref[...], w1[...].T, preferred_element_type=jnp.float32) + b1[0])
    o = jnp.dot(o, w2[...].T, preferred_element_type=jnp.float32) + b2[0]
    out_ref[...] = p * jnp.exp(o)


def _readout_call(x4, o_x, linr_W, linr_b, lino_W1, lino_b1, lino_W2, lino_b2):
    BB = 32
    grid = (B // BB,)
    return pl.pallas_call(
        _readout_body,
        grid=grid,
        in_specs=[
            pl.BlockSpec((BB, S + R, C), lambda i: (i, 0, 0)),
            pl.BlockSpec((BB, 5), lambda i: (i, 0)),
            pl.BlockSpec((7, S + R), lambda i: (0, 0)),
            pl.BlockSpec((1, 7), lambda i: (0, 0)),
            pl.BlockSpec((C, 5), lambda i: (0, 0)),
            pl.BlockSpec((1, C), lambda i: (0, 0)),
            pl.BlockSpec((7, C), lambda i: (0, 0)),
            pl.BlockSpec((1, 7), lambda i: (0, 0)),
        ],
        out_specs=pl.BlockSpec((BB, 7), lambda i: (i, 0)),
        out_shape=jax.ShapeDtypeStruct((B, 7), jnp.float32),
    )(x4.reshape(B, S + R, C), o_x, linr_W, linr_b.reshape(1, 7),
      lino_W1, lino_b1.reshape(1, C), lino_W2, lino_b2.reshape(1, 7))


def kernel(s_x, r_x, edge_index, bn_s_w, bn_s_b, bn_r_w, bn_r_b, lin_s_W,
           lin_r_W, conv_W, conv_b, linr_W, linr_b, lino_W1, lino_b1,
           lino_W2, lino_b2):
    src = edge_index[0]
    dst = edge_index[1]
    src_hi = src + N

    deg0 = _deg_call(dst, 0)
    deg1 = _deg_call(dst, 1)
    s_emb, r_emb, o_x = _prologue_call(
        s_x, r_x, bn_s_w, bn_s_b, bn_r_w, bn_r_b, lin_s_W, lin_r_W)
    x = jnp.concatenate(
        [s_emb.reshape(B, S, C), r_emb.reshape(B, R, C)], axis=1).reshape(N, C)

    dinv, h2 = _dense0_call(x, deg0, deg1, conv_W[0])
    for l in range(NUM_LAYERS):
        h2f = h2.reshape(2 * N, 16)
        agg0 = _agg_call(src, dst, h2f, 0)
        agg1 = _agg_call(src_hi, dst, h2f, 1)
        if l < NUM_LAYERS - 1:
            x, h2 = _dense_call(x, agg0, agg1, dinv, conv_b[l], conv_W[l + 1])
        else:
            x = _dense_last_call(x, agg0, agg1, dinv, conv_b[l])

    return _readout_call(x, o_x, linr_W, linr_b,
                         lino_W1, lino_b1, lino_W2, lino_b2)


# speedup vs baseline: 1.2363x; 1.2363x over previous
import functools

import jax
import jax.numpy as jnp
from jax import lax
from jax.experimental import pallas as pl
from jax.experimental.pallas import tpu as pltpu
from jax.experimental.pallas import tpu_sc as plsc

S, R, C = 38, 340, 24
NUM_LAYERS = 4
EPS = 1e-5
B = 256
N = B * (S + R)
E = N * 16
NS = 16
NC = 2
RPT = N // NS
EPT = E // NS
EPW = E // (NS * NC)
KA = 432
NBUF = 4
KD = 1512
RB = 2016


def _leaky(v):
    return jnp.where(v >= 0, v, 0.2 * v)


RCH = 432


def _deg_body(dst_hbm, zeros_hbm, ones_hbm, out_hbm, dstb, onesb, zbuf, acc):
    c = lax.axis_index("c")
    s = lax.axis_index("s")
    pltpu.sync_copy(zeros_hbm, zbuf)
    pltpu.sync_copy(ones_hbm, onesb)
    for j in range(RPT // RCH):
        pltpu.sync_copy(zbuf, acc.at[pl.ds(s * RPT + j * RCH, RCH)])
    plsc.subcore_barrier()
    base = (c * NS + s) * EPW

    def chunk(i, carry):
        off = base + i * KD
        pltpu.sync_copy(dst_hbm.at[pl.ds(off, KD)], dstb)
        pltpu.sync_copy(onesb, acc.at[dstb], add=True)
        return carry

    lax.fori_loop(0, EPW // KD, chunk, 0)
    plsc.subcore_barrier()
    for j in range(RPT // RCH):
        pltpu.sync_copy(acc.at[pl.ds(s * RPT + j * RCH, RCH)], zbuf)
        pltpu.sync_copy(zbuf, out_hbm.at[pl.ds(c * N + s * RPT + j * RCH, RCH)])


def _agg_body(src2_hbm, dst_hbm, h2_hbm, out_hbm, *sc):
    srcb = sc[0:NBUF]
    dstb = sc[NBUF:2 * NBUF]
    rows = sc[2 * NBUF:3 * NBUF]
    semI = sc[3 * NBUF:4 * NBUF]
    semG = sc[4 * NBUF:5 * NBUF]
    acc = sc[5 * NBUF]
    c = lax.axis_index("c")
    s = lax.axis_index("s")
    for j in range(RPT // RCH):
        pltpu.sync_copy(h2_hbm.at[pl.ds(c * N + s * RPT + j * RCH, RCH)],
                        rows[0].at[pl.ds(0, RCH)])
        pltpu.sync_copy(rows[0].at[pl.ds(0, RCH)],
                        acc.at[pl.ds(s * RPT + j * RCH, RCH)])
    plsc.subcore_barrier()
    base = s * EPT
    nchunk = EPT // KA
    ngrp = nchunk // NBUF

    def fire_idx(i, k):
        off = base + i * KA
        pltpu.async_copy(src2_hbm.at[pl.ds(c * E + off, KA)], srcb[k], semI[k])
        pltpu.async_copy(dst_hbm.at[pl.ds(off, KA)], dstb[k], semI[k])

    def wait_idx(i, k):
        off = base + i * KA
        pltpu.make_async_copy(src2_hbm.at[pl.ds(c * E + off, KA)], srcb[k],
                              semI[k]).wait()
        pltpu.make_async_copy(dst_hbm.at[pl.ds(off, KA)], dstb[k],
                              semI[k]).wait()

    def fire_gather(k):
        pltpu.async_copy(h2_hbm.at[srcb[k]], rows[k], semG[k])

    def wait_gather(k):
        pltpu.make_async_copy(h2_hbm.at[srcb[k]], rows[k], semG[k]).wait()

    for k in range(NBUF):
        fire_idx(k, k)
    wait_idx(0, 0)
    fire_gather(0)

    def grp(g, carry):
        for k in range(NBUF):
            kn = (k + 1) % NBUF

            def issue_next(gg):
                wait_idx(gg * NBUF + k + 1, kn)
                fire_gather(kn)

            if k < NBUF - 1:
                issue_next(g)
            else:
                @pl.when(g < ngrp - 1)
                def _():
                    issue_next(g)
            wait_gather(k)
            pltpu.sync_copy(rows[k], acc.at[dstb[k]], add=True)

            @pl.when(g < ngrp - 1)
            def _():
                fire_idx((g + 1) * NBUF + k, k)
        return carry

    lax.fori_loop(0, ngrp, grp, 0)
    plsc.subcore_barrier()
    for j in range(RPT // RCH):
        pltpu.sync_copy(acc.at[pl.ds(s * RPT + j * RCH, RCH)],
                        rows[0].at[pl.ds(0, RCH)])
        pltpu.sync_copy(rows[0].at[pl.ds(0, RCH)],
                        out_hbm.at[pl.ds(c * N + s * RPT + j * RCH, RCH)])


def _sc_mesh():
    return plsc.VectorSubcoreMesh(core_axis_name="c", subcore_axis_name="s")


def _deg_call(dst):
    fn = pl.kernel(
        _deg_body,
        out_type=jax.ShapeDtypeStruct((NC * N,), jnp.float32),
        mesh=_sc_mesh(),
        scratch_types=[
            pltpu.VMEM((KD,), jnp.int32),
            pltpu.VMEM((KD,), jnp.float32),
            pltpu.VMEM((RCH,), jnp.float32),
            pltpu.VMEM_SHARED((N,), jnp.float32),
        ],
        compiler_params=pltpu.CompilerParams(use_tc_tiling_on_sc=False),
    )
    return fn(dst, jnp.zeros((RCH,), jnp.float32), jnp.ones((KD,), jnp.float32))


def _agg_call(src2, dst, h2):
    fn = pl.kernel(
        _agg_body,
        out_type=jax.ShapeDtypeStruct((NC * N, 16), jnp.float32),
        mesh=_sc_mesh(),
        scratch_types=(
            [pltpu.VMEM((KA,), jnp.int32) for _ in range(2 * NBUF)]
            + [pltpu.VMEM((KA, 16), jnp.float32) for _ in range(NBUF)]
            + [pltpu.SemaphoreType.DMA for _ in range(2 * NBUF)]
            + [pltpu.VMEM_SHARED((N, 16), jnp.float32)]
        ),
        compiler_params=pltpu.CompilerParams(use_tc_tiling_on_sc=False),
    )
    return fn(src2, dst, h2)


PG = 16
SBK = B * S // PG
RBK = B * R // PG


def _stats_body(sx_ref, rx_ref, ss_ref, rs_ref):
    i = pl.program_id(0)
    sx = sx_ref[...]
    rflat = rx_ref[...][:, :3]

    @pl.when(i == 0)
    def _():
        ss_ref[...] = jnp.zeros_like(ss_ref)
        rs_ref[...] = jnp.zeros_like(rs_ref)

    ss_ref[0, :] += jnp.sum(sx, axis=0)
    ss_ref[1, :] += jnp.sum(sx * sx, axis=0)
    rs_ref[0, :] += jnp.sum(rflat, axis=0)
    rs_ref[1, :] += jnp.sum(rflat * rflat, axis=0)


def _stats_call(s_x, r_x):
    return pl.pallas_call(
        _stats_body,
        grid=(PG,),
        in_specs=[
            pl.BlockSpec((SBK, 4), lambda i: (i, 0)),
            pl.BlockSpec((RBK, 8), lambda i: (i, 0)),
        ],
        out_specs=[
            pl.BlockSpec((2, 4), lambda i: (0, 0)),
            pl.BlockSpec((2, 3), lambda i: (0, 0)),
        ],
        out_shape=[
            jax.ShapeDtypeStruct((2, 4), jnp.float32),
            jax.ShapeDtypeStruct((2, 3), jnp.float32),
        ],
    )(s_x, r_x)


def _prologue_body(sx_ref, rx_ref, ss_ref, rs_ref, bnsw, bnsb, bnrw, bnrb,
                   lsw, lrw, s_out, r_out, ox_out):
    ns = float(B * S)
    mu = ss_ref[0, :] / ns
    var = ss_ref[1, :] / ns - mu * mu
    sn = (sx_ref[...] - mu) / jnp.sqrt(var + EPS) * bnsw[0] + bnsb[0]
    s_out[...] = _leaky(jnp.dot(sn, lsw[...].T, preferred_element_type=jnp.float32))

    rx = rx_ref[...]
    nr = float(B * R)
    mu_r = rs_ref[0, :] / nr
    var_r = rs_ref[1, :] / nr - mu_r * mu_r
    rn = (rx[:, :3] - mu_r) / jnp.sqrt(var_r + EPS) * bnrw[0] + bnrb[0]
    r_out[...] = _leaky(jnp.dot(rn, lrw[...].T, preferred_element_type=jnp.float32))

    ox_out[...] = rx.reshape(RBK // R, R, 8)[:, 0, 3:8]


def _prologue_call(s_x, r_x, bn_s_w, bn_s_b, bn_r_w, bn_r_b, lin_s_W, lin_r_W):
    sstats, rstats = _stats_call(s_x, r_x)
    return pl.pallas_call(
        _prologue_body,
        grid=(PG,),
        in_specs=[
            pl.BlockSpec((SBK, 4), lambda i: (i, 0)),
            pl.BlockSpec((RBK, 8), lambda i: (i, 0)),
            pl.BlockSpec((2, 4), lambda i: (0, 0)),
            pl.BlockSpec((2, 3), lambda i: (0, 0)),
            pl.BlockSpec((1, 4), lambda i: (0, 0)),
            pl.BlockSpec((1, 4), lambda i: (0, 0)),
            pl.BlockSpec((1, 3), lambda i: (0, 0)),
            pl.BlockSpec((1, 3), lambda i: (0, 0)),
            pl.BlockSpec((C, 4), lambda i: (0, 0)),
            pl.BlockSpec((C, 3), lambda i: (0, 0)),
        ],
        out_specs=[
            pl.BlockSpec((SBK, C), lambda i: (i, 0)),
            pl.BlockSpec((RBK, C), lambda i: (i, 0)),
            pl.BlockSpec((RBK // R, 5), lambda i: (i, 0)),
        ],
        out_shape=[
            jax.ShapeDtypeStruct((B * S, C), jnp.float32),
            jax.ShapeDtypeStruct((B * R, C), jnp.float32),
            jax.ShapeDtypeStruct((B, 5), jnp.float32),
        ],
    )(s_x, r_x, sstats, rstats, bn_s_w.reshape(1, 4), bn_s_b.reshape(1, 4),
      bn_r_w.reshape(1, 3), bn_r_b.reshape(1, 3), lin_s_W, lin_r_W)


def _split_h(h, h2_ref):
    rb = h.shape[0]
    h2_ref[0] = h[:, :16]
    h2_ref[1] = jnp.concatenate([h[:, 16:24], jnp.zeros((rb, 8), jnp.float32)], axis=1)


def _dense0_body(x_ref, deg0_ref, deg1_ref, w_ref, dinv_ref, h2_ref):
    deg = 1.0 + deg0_ref[:, 0] + deg1_ref[:, 0]
    dinv = lax.rsqrt(deg)
    dinv_ref[:, 0] = dinv
    h = jnp.dot(x_ref[...], w_ref[...].T, preferred_element_type=jnp.float32)
    _split_h(h * dinv[:, None], h2_ref)


def _dense0_call(x0, deg0, deg1, W0):
    grid = (N // RB,)
    return pl.pallas_call(
        _dense0_body,
        grid=grid,
        in_specs=[
            pl.BlockSpec((RB, C), lambda i: (i, 0)),
            pl.BlockSpec((RB, 1), lambda i: (i, 0)),
            pl.BlockSpec((RB, 1), lambda i: (i, 0)),
            pl.BlockSpec((C, C), lambda i: (0, 0)),
        ],
        out_specs=[
            pl.BlockSpec((RB, 1), lambda i: (i, 0)),
            pl.BlockSpec((2, RB, 16), lambda i: (0, i, 0)),
        ],
        out_shape=[
            jax.ShapeDtypeStruct((N, 1), jnp.float32),
            jax.ShapeDtypeStruct((2, N, 16), jnp.float32),
        ],
    )(x0, deg0.reshape(N, 1), deg1.reshape(N, 1), W0)


def _dense_body(x_ref, agg0_ref, agg1_ref, dinv_ref, b_ref, w_ref, xn_ref, h2_ref):
    dinv = dinv_ref[:, 0]
    agg = jnp.concatenate([agg0_ref[...], agg1_ref[:, :8]], axis=1)
    xn = x_ref[...] + _leaky(dinv[:, None] * agg + b_ref[0])
    xn_ref[...] = xn
    h = jnp.dot(xn, w_ref[...].T, preferred_element_type=jnp.float32)
    _split_h(h * dinv[:, None], h2_ref)


def _dense_call(x, agg0, agg1, dinv, b, Wnext):
    grid = (N // RB,)
    return pl.pallas_call(
        _dense_body,
        grid=grid,
        in_specs=[
            pl.BlockSpec((RB, C), lambda i: (i, 0)),
            pl.BlockSpec((RB, 16), lambda i: (i, 0)),
            pl.BlockSpec((RB, 16), lambda i: (i, 0)),
            pl.BlockSpec((RB, 1), lambda i: (i, 0)),
            pl.BlockSpec((1, C), lambda i: (0, 0)),
            pl.BlockSpec((C, C), lambda i: (0, 0)),
        ],
        out_specs=[
            pl.BlockSpec((RB, C), lambda i: (i, 0)),
            pl.BlockSpec((2, RB, 16), lambda i: (0, i, 0)),
        ],
        out_shape=[
            jax.ShapeDtypeStruct((N, C), jnp.float32),
            jax.ShapeDtypeStruct((2, N, 16), jnp.float32),
        ],
    )(x, agg0, agg1, dinv, b.reshape(1, C), Wnext)


def _dense_last_body(x_ref, agg0_ref, agg1_ref, dinv_ref, b_ref, xn_ref):
    dinv = dinv_ref[:, 0]
    agg = jnp.concatenate([agg0_ref[...], agg1_ref[:, :8]], axis=1)
    xn_ref[...] = x_ref[...] + _leaky(dinv[:, None] * agg + b_ref[0])


def _dense_last_call(x, agg0, agg1, dinv, b):
    grid = (N // RB,)
    return pl.pallas_call(
        _dense_last_body,
        grid=grid,
        in_specs=[
            pl.BlockSpec((RB, C), lambda i: (i, 0)),
            pl.BlockSpec((RB, 16), lambda i: (i, 0)),
            pl.BlockSpec((RB, 16), lambda i: (i, 0)),
            pl.BlockSpec((RB, 1), lambda i: (i, 0)),
            pl.BlockSpec((1, C), lambda i: (0, 0)),
        ],
        out_specs=pl.BlockSpec((RB, C), lambda i: (i, 0)),
        out_shape=jax.ShapeDtypeStruct((N, C), jnp.float32),
    )(x, agg0, agg1, dinv, b.reshape(1, C))


def _readout_body(x_ref, ox_ref, lrw, lrb, w1, b1, w2, b2, out_ref):
    xg = jnp.mean(x_ref[...], axis=2)
    logits = jnp.dot(xg, lrw[...].T, preferred_element_type=jnp.float32) + lrb[0]
    exl = jnp.exp(logits)
    p = exl / (jnp.sum(exl, axis=1, keepdims=True) + 1.0)
    o = _leaky(jnp.dot(ox_ref[...], w1[...].T, preferred_element_type=jnp.float32) + b1[0])
    o = jnp.dot(o, w2[...].T, preferred_element_type=jnp.float32) + b2[0]
    out_ref[...] = p * jnp.exp(o)


def _readout_call(x4, o_x, linr_W, linr_b, lino_W1, lino_b1, lino_W2, lino_b2):
    BB = 32
    grid = (B // BB,)
    return pl.pallas_call(
        _readout_body,
        grid=grid,
        in_specs=[
            pl.BlockSpec((BB, S + R, C), lambda i: (i, 0, 0)),
            pl.BlockSpec((BB, 5), lambda i: (i, 0)),
            pl.BlockSpec((7, S + R), lambda i: (0, 0)),
            pl.BlockSpec((1, 7), lambda i: (0, 0)),
            pl.BlockSpec((C, 5), lambda i: (0, 0)),
            pl.BlockSpec((1, C), lambda i: (0, 0)),
            pl.BlockSpec((7, C), lambda i: (0, 0)),
            pl.BlockSpec((1, 7), lambda i: (0, 0)),
        ],
        out_specs=pl.BlockSpec((BB, 7), lambda i: (i, 0)),
        out_shape=jax.ShapeDtypeStruct((B, 7), jnp.float32),
    )(x4.reshape(B, S + R, C), o_x, linr_W, linr_b.reshape(1, 7),
      lino_W1, lino_b1.reshape(1, C), lino_W2, lino_b2.reshape(1, 7))


def kernel(s_x, r_x, edge_index, bn_s_w, bn_s_b, bn_r_w, bn_r_b, lin_s_W,
           lin_r_W, conv_W, conv_b, linr_W, linr_b, lino_W1, lino_b1,
           lino_W2, lino_b2):
    src = edge_index[0]
    dst = edge_index[1]
    src2 = jnp.concatenate([src, src + N])

    degp = _deg_call(dst)
    s_emb, r_emb, o_x = _prologue_call(
        s_x, r_x, bn_s_w, bn_s_b, bn_r_w, bn_r_b, lin_s_W, lin_r_W)
    x = jnp.concatenate(
        [s_emb.reshape(B, S, C), r_emb.reshape(B, R, C)], axis=1).reshape(N, C)

    dinv, h2 = _dense0_call(x, degp[:N], degp[N:], conv_W[0])
    for l in range(NUM_LAYERS):
        agg = _agg_call(src2, dst, h2.reshape(2 * N, 16))
        agg0, agg1 = agg[:N], agg[N:]
        if l < NUM_LAYERS - 1:
            x, h2 = _dense_call(x, agg0, agg1, dinv, conv_b[l], conv_W[l + 1])
        else:
            x = _dense_last_call(x, agg0, agg1, dinv, conv_b[l])

    return _readout_call(x, o_x, linr_W, linr_b,
                         lino_W1, lino_b1, lino_W2, lino_b2)


# speedup vs baseline: 1.6098x; 1.3021x over previous
import functools

import jax
import jax.numpy as jnp
from jax import lax
from jax.experimental import pallas as pl
from jax.experimental.pallas import tpu as pltpu
from jax.experimental.pallas import tpu_sc as plsc

S, R, C = 38, 340, 24
NUM_LAYERS = 4
EPS = 1e-5
B = 256
N = B * (S + R)
E = N * 16
NS = 16
NC = 2
RPT = N // NS
EPT = E // NS
EPW = E // (NS * NC)
KA = 864
NBUF = 2
EPT2 = E // 2 // NS
KD = 1512
RB = 2016


def _leaky(v):
    return jnp.where(v >= 0, v, 0.2 * v)


RCH = 864


def _deg_body(dst_hbm, zeros_hbm, ones_hbm, out_hbm, dstb, onesb, zbuf, acc):
    c = lax.axis_index("c")
    s = lax.axis_index("s")
    pltpu.sync_copy(zeros_hbm, zbuf)
    pltpu.sync_copy(ones_hbm, onesb)
    for j in range(RPT // RCH):
        pltpu.sync_copy(zbuf, acc.at[pl.ds(s * RPT + j * RCH, RCH)])
    plsc.subcore_barrier()
    base = (c * NS + s) * EPW

    def chunk(i, carry):
        off = base + i * KD
        pltpu.sync_copy(dst_hbm.at[pl.ds(off, KD)], dstb)
        pltpu.sync_copy(onesb, acc.at[dstb], add=True)
        return carry

    lax.fori_loop(0, EPW // KD, chunk, 0)
    plsc.subcore_barrier()
    for j in range(RPT // RCH):
        pltpu.sync_copy(acc.at[pl.ds(s * RPT + j * RCH, RCH)], zbuf)
        pltpu.sync_copy(zbuf, out_hbm.at[pl.ds(c * N + s * RPT + j * RCH, RCH)])


def _agg_body(src_hbm, dst_hbm, h2_hbm, zeros_hbm, out_hbm, *sc):
    srcb = sc[0:NBUF]
    dstb = sc[NBUF:2 * NBUF]
    rows = sc[2 * NBUF:3 * NBUF]
    semI = sc[3 * NBUF:4 * NBUF]
    semG = sc[4 * NBUF:5 * NBUF]
    acc = sc[5 * NBUF]
    c = lax.axis_index("c")
    s = lax.axis_index("s")

    @pl.when(c == 0)
    def _():
        for j in range(RPT // RCH):
            pltpu.sync_copy(h2_hbm.at[pl.ds(s * RPT + j * RCH, RCH)],
                            rows[0].at[pl.ds(0, RCH)])
            pltpu.sync_copy(rows[0].at[pl.ds(0, RCH)],
                            acc.at[pl.ds(s * RPT + j * RCH, RCH)])

    @pl.when(c == 1)
    def _():
        pltpu.sync_copy(zeros_hbm, rows[0])
        for j in range(RPT // RCH):
            pltpu.sync_copy(rows[0].at[pl.ds(0, RCH)],
                            acc.at[pl.ds(s * RPT + j * RCH, RCH)])

    plsc.subcore_barrier()
    base = s * EPT2
    nchunk = EPT2 // KA
    ngrp = nchunk // NBUF

    def fire_idx(i, k):
        off = c * (E // 2) + base + i * KA
        pltpu.async_copy(src_hbm.at[pl.ds(off, KA)], srcb[k], semI[k])
        pltpu.async_copy(dst_hbm.at[pl.ds(off, KA)], dstb[k], semI[k])

    def wait_idx(i, k):
        off = c * (E // 2) + base + i * KA
        pltpu.make_async_copy(src_hbm.at[pl.ds(off, KA)], srcb[k],
                              semI[k]).wait()
        pltpu.make_async_copy(dst_hbm.at[pl.ds(off, KA)], dstb[k],
                              semI[k]).wait()

    def fire_gather(k):
        pltpu.async_copy(h2_hbm.at[srcb[k]], rows[k], semG[k])

    def wait_gather(k):
        pltpu.make_async_copy(h2_hbm.at[srcb[k]], rows[k], semG[k]).wait()

    for k in range(NBUF):
        fire_idx(k, k)
    wait_idx(0, 0)
    fire_gather(0)

    def grp(g, carry):
        for k in range(NBUF):
            kn = (k + 1) % NBUF

            def issue_next(gg):
                wait_idx(gg * NBUF + k + 1, kn)
                fire_gather(kn)

            if k < NBUF - 1:
                issue_next(g)
            else:
                @pl.when(g < ngrp - 1)
                def _():
                    issue_next(g)
            wait_gather(k)
            pltpu.sync_copy(rows[k], acc.at[dstb[k]], add=True)

            @pl.when(g < ngrp - 1)
            def _():
                fire_idx((g + 1) * NBUF + k, k)
        return carry

    lax.fori_loop(0, ngrp, grp, 0)
    plsc.subcore_barrier()
    for j in range(RPT // RCH):
        pltpu.sync_copy(acc.at[pl.ds(s * RPT + j * RCH, RCH)],
                        rows[0].at[pl.ds(0, RCH)])
        pltpu.sync_copy(rows[0].at[pl.ds(0, RCH)],
                        out_hbm.at[pl.ds(c * N + s * RPT + j * RCH, RCH)])


def _agg_call(src, dst, h2):
    fn = pl.kernel(
        _agg_body,
        out_type=jax.ShapeDtypeStruct((NC * N, 32), jnp.bfloat16),
        mesh=_sc_mesh(),
        scratch_types=(
            [pltpu.VMEM((KA,), jnp.int32) for _ in range(2 * NBUF)]
            + [pltpu.VMEM((KA, 32), jnp.bfloat16) for _ in range(NBUF)]
            + [pltpu.SemaphoreType.DMA for _ in range(2 * NBUF)]
            + [pltpu.VMEM_SHARED((N, 32), jnp.bfloat16)]
        ),
        compiler_params=pltpu.CompilerParams(use_tc_tiling_on_sc=False),
    )
    return fn(src, dst, h2, jnp.zeros((RCH, 32), jnp.bfloat16))


def _sc_mesh():
    return plsc.VectorSubcoreMesh(core_axis_name="c", subcore_axis_name="s")


def _deg_call(dst):
    fn = pl.kernel(
        _deg_body,
        out_type=jax.ShapeDtypeStruct((NC * N,), jnp.float32),
        mesh=_sc_mesh(),
        scratch_types=[
            pltpu.VMEM((KD,), jnp.int32),
            pltpu.VMEM((KD,), jnp.float32),
            pltpu.VMEM((RCH,), jnp.float32),
            pltpu.VMEM_SHARED((N,), jnp.float32),
        ],
        compiler_params=pltpu.CompilerParams(use_tc_tiling_on_sc=False),
    )
    return fn(dst, jnp.zeros((RCH,), jnp.float32), jnp.ones((KD,), jnp.float32))


PG = 16
SBK = B * S // PG
RBK = B * R // PG


def _stats_body(sx_ref, rx_ref, ss_ref, rs_ref):
    i = pl.program_id(0)
    sx = sx_ref[...]
    rflat = rx_ref[...][:, :3]

    @pl.when(i == 0)
    def _():
        ss_ref[...] = jnp.zeros_like(ss_ref)
        rs_ref[...] = jnp.zeros_like(rs_ref)

    ss_ref[0, :] += jnp.sum(sx, axis=0)
    ss_ref[1, :] += jnp.sum(sx * sx, axis=0)
    rs_ref[0, :] += jnp.sum(rflat, axis=0)
    rs_ref[1, :] += jnp.sum(rflat * rflat, axis=0)


def _stats_call(s_x, r_x):
    return pl.pallas_call(
        _stats_body,
        grid=(PG,),
        in_specs=[
            pl.BlockSpec((SBK, 4), lambda i: (i, 0)),
            pl.BlockSpec((RBK, 8), lambda i: (i, 0)),
        ],
        out_specs=[
            pl.BlockSpec((2, 4), lambda i: (0, 0)),
            pl.BlockSpec((2, 3), lambda i: (0, 0)),
        ],
        out_shape=[
            jax.ShapeDtypeStruct((2, 4), jnp.float32),
            jax.ShapeDtypeStruct((2, 3), jnp.float32),
        ],
    )(s_x, r_x)


def _prologue_body(sx_ref, rx_ref, ss_ref, rs_ref, bnsw, bnsb, bnrw, bnrb,
                   lsw, lrw, s_out, r_out, ox_out):
    ns = float(B * S)
    mu = ss_ref[0, :] / ns
    var = ss_ref[1, :] / ns - mu * mu
    sn = (sx_ref[...] - mu) / jnp.sqrt(var + EPS) * bnsw[0] + bnsb[0]
    s_out[...] = _leaky(jnp.dot(sn, lsw[...].T, preferred_element_type=jnp.float32))

    rx = rx_ref[...]
    nr = float(B * R)
    mu_r = rs_ref[0, :] / nr
    var_r = rs_ref[1, :] / nr - mu_r * mu_r
    rn = (rx[:, :3] - mu_r) / jnp.sqrt(var_r + EPS) * bnrw[0] + bnrb[0]
    r_out[...] = _leaky(jnp.dot(rn, lrw[...].T, preferred_element_type=jnp.float32))

    ox_out[...] = rx.reshape(RBK // R, R, 8)[:, 0, 3:8]


def _prologue_call(s_x, r_x, bn_s_w, bn_s_b, bn_r_w, bn_r_b, lin_s_W, lin_r_W):
    sstats, rstats = _stats_call(s_x, r_x)
    return pl.pallas_call(
        _prologue_body,
        grid=(PG,),
        in_specs=[
            pl.BlockSpec((SBK, 4), lambda i: (i, 0)),
            pl.BlockSpec((RBK, 8), lambda i: (i, 0)),
            pl.BlockSpec((2, 4), lambda i: (0, 0)),
            pl.BlockSpec((2, 3), lambda i: (0, 0)),
            pl.BlockSpec((1, 4), lambda i: (0, 0)),
            pl.BlockSpec((1, 4), lambda i: (0, 0)),
            pl.BlockSpec((1, 3), lambda i: (0, 0)),
            pl.BlockSpec((1, 3), lambda i: (0, 0)),
            pl.BlockSpec((C, 4), lambda i: (0, 0)),
            pl.BlockSpec((C, 3), lambda i: (0, 0)),
        ],
        out_specs=[
            pl.BlockSpec((SBK, C), lambda i: (i, 0)),
            pl.BlockSpec((RBK, C), lambda i: (i, 0)),
            pl.BlockSpec((RBK // R, 5), lambda i: (i, 0)),
        ],
        out_shape=[
            jax.ShapeDtypeStruct((B * S, C), jnp.float32),
            jax.ShapeDtypeStruct((B * R, C), jnp.float32),
            jax.ShapeDtypeStruct((B, 5), jnp.float32),
        ],
    )(s_x, r_x, sstats, rstats, bn_s_w.reshape(1, 4), bn_s_b.reshape(1, 4),
      bn_r_w.reshape(1, 3), bn_r_b.reshape(1, 3), lin_s_W, lin_r_W)


def _split_h(h, h2_ref):
    rb = h.shape[0]
    h2_ref[...] = jnp.concatenate(
        [h, jnp.zeros((rb, 8), jnp.float32)], axis=1).astype(jnp.bfloat16)


def _dense0_body(x_ref, deg0_ref, deg1_ref, w_ref, dinv_ref, h2_ref):
    deg = 1.0 + deg0_ref[:, 0] + deg1_ref[:, 0]
    dinv = lax.rsqrt(deg)
    dinv_ref[:, 0] = dinv
    h = jnp.dot(x_ref[...], w_ref[...].T, preferred_element_type=jnp.float32)
    _split_h(h * dinv[:, None], h2_ref)


def _dense0_call(x0, deg0, deg1, W0):
    grid = (N // RB,)
    return pl.pallas_call(
        _dense0_body,
        grid=grid,
        in_specs=[
            pl.BlockSpec((RB, C), lambda i: (i, 0)),
            pl.BlockSpec((RB, 1), lambda i: (i, 0)),
            pl.BlockSpec((RB, 1), lambda i: (i, 0)),
            pl.BlockSpec((C, C), lambda i: (0, 0)),
        ],
        out_specs=[
            pl.BlockSpec((RB, 1), lambda i: (i, 0)),
            pl.BlockSpec((RB, 32), lambda i: (i, 0)),
        ],
        out_shape=[
            jax.ShapeDtypeStruct((N, 1), jnp.float32),
            jax.ShapeDtypeStruct((N, 32), jnp.bfloat16),
        ],
    )(x0, deg0.reshape(N, 1), deg1.reshape(N, 1), W0)


def _dense_body(x_ref, agg0_ref, agg1_ref, dinv_ref, b_ref, w_ref, xn_ref, h2_ref):
    dinv = dinv_ref[:, 0]
    agg = (agg0_ref[...].astype(jnp.float32)
           + agg1_ref[...].astype(jnp.float32))[:, :C]
    xn = x_ref[...] + _leaky(dinv[:, None] * agg + b_ref[0])
    xn_ref[...] = xn
    h = jnp.dot(xn, w_ref[...].T, preferred_element_type=jnp.float32)
    _split_h(h * dinv[:, None], h2_ref)


def _dense_call(x, agg0, agg1, dinv, b, Wnext):
    grid = (N // RB,)
    return pl.pallas_call(
        _dense_body,
        grid=grid,
        in_specs=[
            pl.BlockSpec((RB, C), lambda i: (i, 0)),
            pl.BlockSpec((RB, 32), lambda i: (i, 0)),
            pl.BlockSpec((RB, 32), lambda i: (i, 0)),
            pl.BlockSpec((RB, 1), lambda i: (i, 0)),
            pl.BlockSpec((1, C), lambda i: (0, 0)),
            pl.BlockSpec((C, C), lambda i: (0, 0)),
        ],
        out_specs=[
            pl.BlockSpec((RB, C), lambda i: (i, 0)),
            pl.BlockSpec((RB, 32), lambda i: (i, 0)),
        ],
        out_shape=[
            jax.ShapeDtypeStruct((N, C), jnp.float32),
            jax.ShapeDtypeStruct((N, 32), jnp.bfloat16),
        ],
    )(x, agg0, agg1, dinv, b.reshape(1, C), Wnext)


def _dense_last_body(x_ref, agg0_ref, agg1_ref, dinv_ref, b_ref, xn_ref):
    dinv = dinv_ref[:, 0]
    agg = (agg0_ref[...].astype(jnp.float32)
           + agg1_ref[...].astype(jnp.float32))[:, :C]
    xn_ref[...] = x_ref[...] + _leaky(dinv[:, None] * agg + b_ref[0])


def _dense_last_call(x, agg0, agg1, dinv, b):
    grid = (N // RB,)
    return pl.pallas_call(
        _dense_last_body,
        grid=grid,
        in_specs=[
            pl.BlockSpec((RB, C), lambda i: (i, 0)),
            pl.BlockSpec((RB, 32), lambda i: (i, 0)),
            pl.BlockSpec((RB, 32), lambda i: (i, 0)),
            pl.BlockSpec((RB, 1), lambda i: (i, 0)),
            pl.BlockSpec((1, C), lambda i: (0, 0)),
        ],
        out_specs=pl.BlockSpec((RB, C), lambda i: (i, 0)),
        out_shape=jax.ShapeDtypeStruct((N, C), jnp.float32),
    )(x, agg0, agg1, dinv, b.reshape(1, C))


def _readout_body(x_ref, ox_ref, lrw, lrb, w1, b1, w2, b2, out_ref):
    xg = jnp.mean(x_ref[...], axis=2)
    logits = jnp.dot(xg, lrw[...].T, preferred_element_type=jnp.float32) + lrb[0]
    exl = jnp.exp(logits)
    p = exl / (jnp.sum(exl, axis=1, keepdims=True) + 1.0)
    o = _leaky(jnp.dot(ox_ref[...], w1[...].T, preferred_element_type=jnp.float32) + b1[0])
    o = jnp.dot(o, w2[...].T, preferred_element_type=jnp.float32) + b2[0]
    out_ref[...] = p * jnp.exp(o)


def _readout_call(x4, o_x, linr_W, linr_b, lino_W1, lino_b1, lino_W2, lino_b2):
    BB = 32
    grid = (B // BB,)
    return pl.pallas_call(
        _readout_body,
        grid=grid,
        in_specs=[
            pl.BlockSpec((BB, S + R, C), lambda i: (i, 0, 0)),
            pl.BlockSpec((BB, 5), lambda i: (i, 0)),
            pl.BlockSpec((7, S + R), lambda i: (0, 0)),
            pl.BlockSpec((1, 7), lambda i: (0, 0)),
            pl.BlockSpec((C, 5), lambda i: (0, 0)),
            pl.BlockSpec((1, C), lambda i: (0, 0)),
            pl.BlockSpec((7, C), lambda i: (0, 0)),
            pl.BlockSpec((1, 7), lambda i: (0, 0)),
        ],
        out_specs=pl.BlockSpec((BB, 7), lambda i: (i, 0)),
        out_shape=jax.ShapeDtypeStruct((B, 7), jnp.float32),
    )(x4.reshape(B, S + R, C), o_x, linr_W, linr_b.reshape(1, 7),
      lino_W1, lino_b1.reshape(1, C), lino_W2, lino_b2.reshape(1, 7))


def kernel(s_x, r_x, edge_index, bn_s_w, bn_s_b, bn_r_w, bn_r_b, lin_s_W,
           lin_r_W, conv_W, conv_b, linr_W, linr_b, lino_W1, lino_b1,
           lino_W2, lino_b2):
    src = edge_index[0]
    dst = edge_index[1]

    degp = _deg_call(dst)
    s_emb, r_emb, o_x = _prologue_call(
        s_x, r_x, bn_s_w, bn_s_b, bn_r_w, bn_r_b, lin_s_W, lin_r_W)
    x = jnp.concatenate(
        [s_emb.reshape(B, S, C), r_emb.reshape(B, R, C)], axis=1).reshape(N, C)

    dinv, h2 = _dense0_call(x, degp[:N], degp[N:], conv_W[0])
    for l in range(NUM_LAYERS):
        agg = _agg_call(src, dst, h2)
        agg0, agg1 = agg[:N], agg[N:]
        if l < NUM_LAYERS - 1:
            x, h2 = _dense_call(x, agg0, agg1, dinv, conv_b[l], conv_W[l + 1])
        else:
            x = _dense_last_call(x, agg0, agg1, dinv, conv_b[l])

    return _readout_call(x, o_x, linr_W, linr_b,
                         lino_W1, lino_b1, lino_W2, lino_b2)


# speedup vs baseline: 1.6221x; 1.0076x over previous
import functools

import jax
import jax.numpy as jnp
from jax import lax
from jax.experimental import pallas as pl
from jax.experimental.pallas import tpu as pltpu
from jax.experimental.pallas import tpu_sc as plsc

S, R, C = 38, 340, 24
NUM_LAYERS = 4
EPS = 1e-5
B = 256
N = B * (S + R)
E = N * 16
NS = 16
NC = 2
RPT = N // NS
EPT = E // NS
EPW = E // (NS * NC)
KA = 864
NBUF = 2
EPT2 = E // 2 // NS
KD = 1512
RB = 2016


def _leaky(v):
    return jnp.where(v >= 0, v, 0.2 * v)


RCH = 864


def _deg_body(dst_hbm, zeros_hbm, ones_hbm, out_hbm, dstb, onesb, zbuf, acc):
    c = lax.axis_index("c")
    s = lax.axis_index("s")
    pltpu.sync_copy(zeros_hbm, zbuf)
    pltpu.sync_copy(ones_hbm, onesb)
    for j in range(RPT // RCH):
        pltpu.sync_copy(zbuf, acc.at[pl.ds(s * RPT + j * RCH, RCH)])
    plsc.subcore_barrier()
    base = (c * NS + s) * EPW

    def chunk(i, carry):
        off = base + i * KD
        pltpu.sync_copy(dst_hbm.at[pl.ds(off, KD)], dstb)
        pltpu.sync_copy(onesb, acc.at[dstb], add=True)
        return carry

    lax.fori_loop(0, EPW // KD, chunk, 0)
    plsc.subcore_barrier()
    for j in range(RPT // RCH):
        pltpu.sync_copy(acc.at[pl.ds(s * RPT + j * RCH, RCH)], zbuf)
        pltpu.sync_copy(zbuf, out_hbm.at[pl.ds(c * N + s * RPT + j * RCH, RCH)])


def _agg_body(src_hbm, dst_hbm, h2_hbm, zeros_hbm, out_hbm, *sc):
    srcb = sc[0:NBUF]
    dstb = sc[NBUF:2 * NBUF]
    rows = sc[2 * NBUF:3 * NBUF]
    semI = sc[3 * NBUF:4 * NBUF]
    semG = sc[4 * NBUF:5 * NBUF]
    acc = sc[5 * NBUF]
    c = lax.axis_index("c")
    s = lax.axis_index("s")

    pltpu.sync_copy(zeros_hbm, rows[0])
    nz = RPT // RCH
    for j in range(nz):
        pltpu.async_copy(rows[0].at[pl.ds(0, RCH)],
                         acc.at[pl.ds(s * RPT + j * RCH, RCH)], semG[0])
    for j in range(nz):
        pltpu.make_async_copy(rows[0].at[pl.ds(0, RCH)],
                              acc.at[pl.ds(s * RPT + j * RCH, RCH)],
                              semG[0]).wait()

    plsc.subcore_barrier()
    base = s * EPT2
    nchunk = EPT2 // KA
    ngrp = nchunk // NBUF

    def fire_idx(i, k):
        off = c * (E // 2) + base + i * KA
        pltpu.async_copy(src_hbm.at[pl.ds(off, KA)], srcb[k], semI[k])
        pltpu.async_copy(dst_hbm.at[pl.ds(off, KA)], dstb[k], semI[k])

    def wait_idx(i, k):
        off = c * (E // 2) + base + i * KA
        pltpu.make_async_copy(src_hbm.at[pl.ds(off, KA)], srcb[k],
                              semI[k]).wait()
        pltpu.make_async_copy(dst_hbm.at[pl.ds(off, KA)], dstb[k],
                              semI[k]).wait()

    def fire_gather(k):
        pltpu.async_copy(h2_hbm.at[srcb[k]], rows[k], semG[k])

    def wait_gather(k):
        pltpu.make_async_copy(h2_hbm.at[srcb[k]], rows[k], semG[k]).wait()

    for k in range(NBUF):
        fire_idx(k, k)
    wait_idx(0, 0)
    fire_gather(0)

    def grp(g, carry):
        for k in range(NBUF):
            kn = (k + 1) % NBUF

            def issue_next(gg):
                wait_idx(gg * NBUF + k + 1, kn)
                fire_gather(kn)

            if k < NBUF - 1:
                issue_next(g)
            else:
                @pl.when(g < ngrp - 1)
                def _():
                    issue_next(g)
            wait_gather(k)
            pltpu.sync_copy(rows[k], acc.at[dstb[k]], add=True)

            @pl.when(g < ngrp - 1)
            def _():
                fire_idx((g + 1) * NBUF + k, k)
        return carry

    lax.fori_loop(0, ngrp, grp, 0)
    plsc.subcore_barrier()

    def wb_ld(j, start):
        b = j % 2
        d = pltpu.async_copy if start else pltpu.make_async_copy
        r = d(acc.at[pl.ds(s * RPT + j * RCH, RCH)], rows[b], semG[b])
        if not start:
            r.wait()

    def wb_st(j, start):
        b = j % 2
        d = pltpu.async_copy if start else pltpu.make_async_copy
        r = d(rows[b], out_hbm.at[pl.ds(c * N + s * RPT + j * RCH, RCH)],
              semI[b])
        if not start:
            r.wait()

    nz2 = RPT // RCH
    wb_ld(0, True)
    wb_ld(1, True)
    for j in range(nz2):
        wb_ld(j, False)
        wb_st(j, True)
        if j + 2 < nz2:
            wb_st(j, False)
            wb_ld(j + 2, True)
        else:
            wb_st(j, False)


def _agg_call(src, dst, h2):
    fn = pl.kernel(
        _agg_body,
        out_type=jax.ShapeDtypeStruct((NC * N, 32), jnp.bfloat16),
        mesh=_sc_mesh(),
        scratch_types=(
            [pltpu.VMEM((KA,), jnp.int32) for _ in range(2 * NBUF)]
            + [pltpu.VMEM((KA, 32), jnp.bfloat16) for _ in range(NBUF)]
            + [pltpu.SemaphoreType.DMA for _ in range(2 * NBUF)]
            + [pltpu.VMEM_SHARED((N, 32), jnp.bfloat16)]
        ),
        compiler_params=pltpu.CompilerParams(use_tc_tiling_on_sc=False),
    )
    return fn(src, dst, h2, jnp.zeros((RCH, 32), jnp.bfloat16))


def _sc_mesh():
    return plsc.VectorSubcoreMesh(core_axis_name="c", subcore_axis_name="s")


def _deg_call(dst):
    fn = pl.kernel(
        _deg_body,
        out_type=jax.ShapeDtypeStruct((NC * N,), jnp.float32),
        mesh=_sc_mesh(),
        scratch_types=[
            pltpu.VMEM((KD,), jnp.int32),
            pltpu.VMEM((KD,), jnp.float32),
            pltpu.VMEM((RCH,), jnp.float32),
            pltpu.VMEM_SHARED((N,), jnp.float32),
        ],
        compiler_params=pltpu.CompilerParams(use_tc_tiling_on_sc=False),
    )
    return fn(dst, jnp.zeros((RCH,), jnp.float32), jnp.ones((KD,), jnp.float32))


PG = 16
SBK = B * S // PG
RBK = B * R // PG


def _stats_body(sx_ref, rx_ref, ss_ref, rs_ref):
    i = pl.program_id(0)
    sx = sx_ref[...]
    rflat = rx_ref[...][:, :3]

    @pl.when(i == 0)
    def _():
        ss_ref[...] = jnp.zeros_like(ss_ref)
        rs_ref[...] = jnp.zeros_like(rs_ref)

    ss_ref[0, :] += jnp.sum(sx, axis=0)
    ss_ref[1, :] += jnp.sum(sx * sx, axis=0)
    rs_ref[0, :] += jnp.sum(rflat, axis=0)
    rs_ref[1, :] += jnp.sum(rflat * rflat, axis=0)


def _stats_call(s_x, r_x):
    return pl.pallas_call(
        _stats_body,
        grid=(PG,),
        in_specs=[
            pl.BlockSpec((SBK, 4), lambda i: (i, 0)),
            pl.BlockSpec((RBK, 8), lambda i: (i, 0)),
        ],
        out_specs=[
            pl.BlockSpec((2, 4), lambda i: (0, 0)),
            pl.BlockSpec((2, 3), lambda i: (0, 0)),
        ],
        out_shape=[
            jax.ShapeDtypeStruct((2, 4), jnp.float32),
            jax.ShapeDtypeStruct((2, 3), jnp.float32),
        ],
    )(s_x, r_x)


def _prologue_body(sx_ref, rx_ref, ss_ref, rs_ref, bnsw, bnsb, bnrw, bnrb,
                   lsw, lrw, s_out, r_out, ox_out):
    ns = float(B * S)
    mu = ss_ref[0, :] / ns
    var = ss_ref[1, :] / ns - mu * mu
    sn = (sx_ref[...] - mu) / jnp.sqrt(var + EPS) * bnsw[0] + bnsb[0]
    s_out[...] = _leaky(jnp.dot(sn, lsw[...].T, preferred_element_type=jnp.float32))

    rx = rx_ref[...]
    nr = float(B * R)
    mu_r = rs_ref[0, :] / nr
    var_r = rs_ref[1, :] / nr - mu_r * mu_r
    rn = (rx[:, :3] - mu_r) / jnp.sqrt(var_r + EPS) * bnrw[0] + bnrb[0]
    r_out[...] = _leaky(jnp.dot(rn, lrw[...].T, preferred_element_type=jnp.float32))

    ox_out[...] = rx.reshape(RBK // R, R, 8)[:, 0, 3:8]


def _prologue_call(s_x, r_x, bn_s_w, bn_s_b, bn_r_w, bn_r_b, lin_s_W, lin_r_W):
    sstats, rstats = _stats_call(s_x, r_x)
    return pl.pallas_call(
        _prologue_body,
        grid=(PG,),
        in_specs=[
            pl.BlockSpec((SBK, 4), lambda i: (i, 0)),
            pl.BlockSpec((RBK, 8), lambda i: (i, 0)),
            pl.BlockSpec((2, 4), lambda i: (0, 0)),
            pl.BlockSpec((2, 3), lambda i: (0, 0)),
            pl.BlockSpec((1, 4), lambda i: (0, 0)),
            pl.BlockSpec((1, 4), lambda i: (0, 0)),
            pl.BlockSpec((1, 3), lambda i: (0, 0)),
            pl.BlockSpec((1, 3), lambda i: (0, 0)),
            pl.BlockSpec((C, 4), lambda i: (0, 0)),
            pl.BlockSpec((C, 3), lambda i: (0, 0)),
        ],
        out_specs=[
            pl.BlockSpec((SBK, C), lambda i: (i, 0)),
            pl.BlockSpec((RBK, C), lambda i: (i, 0)),
            pl.BlockSpec((RBK // R, 5), lambda i: (i, 0)),
        ],
        out_shape=[
            jax.ShapeDtypeStruct((B * S, C), jnp.float32),
            jax.ShapeDtypeStruct((B * R, C), jnp.float32),
            jax.ShapeDtypeStruct((B, 5), jnp.float32),
        ],
    )(s_x, r_x, sstats, rstats, bn_s_w.reshape(1, 4), bn_s_b.reshape(1, 4),
      bn_r_w.reshape(1, 3), bn_r_b.reshape(1, 3), lin_s_W, lin_r_W)


def _split_h(h, h2_ref):
    rb = h.shape[0]
    h2_ref[...] = jnp.concatenate(
        [h, jnp.zeros((rb, 8), jnp.float32)], axis=1).astype(jnp.bfloat16)


def _dense0_body(x_ref, deg0_ref, deg1_ref, w_ref, dinv_ref, h2_ref):
    deg = 1.0 + deg0_ref[:, 0] + deg1_ref[:, 0]
    dinv = lax.rsqrt(deg)
    dinv_ref[:, 0] = dinv
    h = jnp.dot(x_ref[...], w_ref[...].T, preferred_element_type=jnp.float32)
    _split_h(h * dinv[:, None], h2_ref)


def _dense0_call(x0, deg0, deg1, W0):
    grid = (N // RB,)
    return pl.pallas_call(
        _dense0_body,
        grid=grid,
        in_specs=[
            pl.BlockSpec((RB, C), lambda i: (i, 0)),
            pl.BlockSpec((RB, 1), lambda i: (i, 0)),
            pl.BlockSpec((RB, 1), lambda i: (i, 0)),
            pl.BlockSpec((C, C), lambda i: (0, 0)),
        ],
        out_specs=[
            pl.BlockSpec((RB, 1), lambda i: (i, 0)),
            pl.BlockSpec((RB, 32), lambda i: (i, 0)),
        ],
        out_shape=[
            jax.ShapeDtypeStruct((N, 1), jnp.float32),
            jax.ShapeDtypeStruct((N, 32), jnp.bfloat16),
        ],
    )(x0, deg0.reshape(N, 1), deg1.reshape(N, 1), W0)


def _dense_body(x_ref, agg0_ref, agg1_ref, hp_ref, dinv_ref, b_ref, w_ref,
                xn_ref, h2_ref):
    dinv = dinv_ref[:, 0]
    agg = (agg0_ref[...].astype(jnp.float32)
           + agg1_ref[...].astype(jnp.float32)
           + hp_ref[...].astype(jnp.float32))[:, :C]
    xn = x_ref[...] + _leaky(dinv[:, None] * agg + b_ref[0])
    xn_ref[...] = xn
    h = jnp.dot(xn, w_ref[...].T, preferred_element_type=jnp.float32)
    _split_h(h * dinv[:, None], h2_ref)


def _dense_call(x, agg0, agg1, hp, dinv, b, Wnext):
    grid = (N // RB,)
    return pl.pallas_call(
        _dense_body,
        grid=grid,
        in_specs=[
            pl.BlockSpec((RB, C), lambda i: (i, 0)),
            pl.BlockSpec((RB, 32), lambda i: (i, 0)),
            pl.BlockSpec((RB, 32), lambda i: (i, 0)),
            pl.BlockSpec((RB, 32), lambda i: (i, 0)),
            pl.BlockSpec((RB, 1), lambda i: (i, 0)),
            pl.BlockSpec((1, C), lambda i: (0, 0)),
            pl.BlockSpec((C, C), lambda i: (0, 0)),
        ],
        out_specs=[
            pl.BlockSpec((RB, C), lambda i: (i, 0)),
            pl.BlockSpec((RB, 32), lambda i: (i, 0)),
        ],
        out_shape=[
            jax.ShapeDtypeStruct((N, C), jnp.float32),
            jax.ShapeDtypeStruct((N, 32), jnp.bfloat16),
        ],
    )(x, agg0, agg1, hp, dinv, b.reshape(1, C), Wnext)


def _dense_last_body(x_ref, agg0_ref, agg1_ref, hp_ref, dinv_ref, b_ref, xn_ref):
    dinv = dinv_ref[:, 0]
    agg = (agg0_ref[...].astype(jnp.float32)
           + agg1_ref[...].astype(jnp.float32)
           + hp_ref[...].astype(jnp.float32))[:, :C]
    xn_ref[...] = x_ref[...] + _leaky(dinv[:, None] * agg + b_ref[0])


def _dense_last_call(x, agg0, agg1, hp, dinv, b):
    grid = (N // RB,)
    return pl.pallas_call(
        _dense_last_body,
        grid=grid,
        in_specs=[
            pl.BlockSpec((RB, C), lambda i: (i, 0)),
            pl.BlockSpec((RB, 32), lambda i: (i, 0)),
            pl.BlockSpec((RB, 32), lambda i: (i, 0)),
            pl.BlockSpec((RB, 32), lambda i: (i, 0)),
            pl.BlockSpec((RB, 1), lambda i: (i, 0)),
            pl.BlockSpec((1, C), lambda i: (0, 0)),
        ],
        out_specs=pl.BlockSpec((RB, C), lambda i: (i, 0)),
        out_shape=jax.ShapeDtypeStruct((N, C), jnp.float32),
    )(x, agg0, agg1, hp, dinv, b.reshape(1, C))


def _readout_body(x_ref, ox_ref, lrw, lrb, w1, b1, w2, b2, out_ref):
    xg = jnp.mean(x_ref[...], axis=2)
    logits = jnp.dot(xg, lrw[...].T, preferred_element_type=jnp.float32) + lrb[0]
    exl = jnp.exp(logits)
    p = exl / (jnp.sum(exl, axis=1, keepdims=True) + 1.0)
    o = _leaky(jnp.dot(ox_ref[...], w1[...].T, preferred_element_type=jnp.float32) + b1[0])
    o = jnp.dot(o, w2[...].T, preferred_element_type=jnp.float32) + b2[0]
    out_ref[...] = p * jnp.exp(o)


def _readout_call(x4, o_x, linr_W, linr_b, lino_W1, lino_b1, lino_W2, lino_b2):
    BB = 32
    grid = (B // BB,)
    return pl.pallas_call(
        _readout_body,
        grid=grid,
        in_specs=[
            pl.BlockSpec((BB, S + R, C), lambda i: (i, 0, 0)),
            pl.BlockSpec((BB, 5), lambda i: (i, 0)),
            pl.BlockSpec((7, S + R), lambda i: (0, 0)),
            pl.BlockSpec((1, 7), lambda i: (0, 0)),
            pl.BlockSpec((C, 5), lambda i: (0, 0)),
            pl.BlockSpec((1, C), lambda i: (0, 0)),
            pl.BlockSpec((7, C), lambda i: (0, 0)),
            pl.BlockSpec((1, 7), lambda i: (0, 0)),
        ],
        out_specs=pl.BlockSpec((BB, 7), lambda i: (i, 0)),
        out_shape=jax.ShapeDtypeStruct((B, 7), jnp.float32),
    )(x4.reshape(B, S + R, C), o_x, linr_W, linr_b.reshape(1, 7),
      lino_W1, lino_b1.reshape(1, C), lino_W2, lino_b2.reshape(1, 7))


def kernel(s_x, r_x, edge_index, bn_s_w, bn_s_b, bn_r_w, bn_r_b, lin_s_W,
           lin_r_W, conv_W, conv_b, linr_W, linr_b, lino_W1, lino_b1,
           lino_W2, lino_b2):
    src = edge_index[0]
    dst = edge_index[1]

    degp = _deg_call(dst)
    s_emb, r_emb, o_x = _prologue_call(
        s_x, r_x, bn_s_w, bn_s_b, bn_r_w, bn_r_b, lin_s_W, lin_r_W)
    x = jnp.concatenate(
        [s_emb.reshape(B, S, C), r_emb.reshape(B, R, C)], axis=1).reshape(N, C)

    dinv, h2 = _dense0_call(x, degp[:N], degp[N:], conv_W[0])
    for l in range(NUM_LAYERS):
        agg = _agg_call(src, dst, h2)
        agg0, agg1 = agg[:N], agg[N:]
        if l < NUM_LAYERS - 1:
            x, h2 = _dense_call(x, agg0, agg1, h2, dinv, conv_b[l],
                                conv_W[l + 1])
        else:
            x = _dense_last_call(x, agg0, agg1, h2, dinv, conv_b[l])

    return _readout_call(x, o_x, linr_W, linr_b,
                         lino_W1, lino_b1, lino_W2, lino_b2)


# speedup vs baseline: 1.6845x; 1.0385x over previous
import functools

import jax
import jax.numpy as jnp
from jax import lax
from jax.experimental import pallas as pl
from jax.experimental.pallas import tpu as pltpu
from jax.experimental.pallas import tpu_sc as plsc

S, R, C = 38, 340, 24
NUM_LAYERS = 4
EPS = 1e-5
B = 256
N = B * (S + R)
E = N * 16
NS = 16
NC = 2
RPT = N // NS
EPT = E // NS
EPW = E // (NS * NC)
KA = 864
NBUF = 2
KD = 1512
RB = 2016


def _leaky(v):
    return jnp.where(v >= 0, v, 0.2 * v)


RCH = 864


def _deg_body(dst_hbm, zeros_hbm, ones_hbm, out_hbm, dstb, onesb, zbuf, acc):
    s = lax.axis_index("s")
    pltpu.sync_copy(zeros_hbm, zbuf)
    pltpu.sync_copy(ones_hbm, onesb)
    for j in range(RPT // RCH):
        pltpu.sync_copy(zbuf, acc.at[pl.ds(s * RPT + j * RCH, RCH)])
    plsc.subcore_barrier()
    base = s * (E // NS)

    def chunk(i, carry):
        off = base + i * KD
        pltpu.sync_copy(dst_hbm.at[pl.ds(off, KD)], dstb)
        pltpu.sync_copy(onesb, acc.at[dstb], add=True)
        return carry

    lax.fori_loop(0, E // NS // KD, chunk, 0)
    plsc.subcore_barrier()
    for j in range(RPT // RCH):
        pltpu.sync_copy(acc.at[pl.ds(s * RPT + j * RCH, RCH)], zbuf)
        pltpu.sync_copy(zbuf, out_hbm.at[pl.ds(s * RPT + j * RCH, RCH)])


def _agg_body(src_hbm, dst_hbm, h2_hbm, zeros_hbm, out_hbm, *sc):
    srcb = sc[0:NBUF]
    dstb = sc[NBUF:2 * NBUF]
    rows = sc[2 * NBUF:3 * NBUF]
    semI = sc[3 * NBUF:4 * NBUF]
    semG = sc[4 * NBUF:5 * NBUF]
    acc = sc[5 * NBUF]
    s = lax.axis_index("s")

    pltpu.sync_copy(zeros_hbm, rows[0])
    nz = RPT // RCH
    for j in range(nz):
        pltpu.async_copy(rows[0].at[pl.ds(0, RCH)],
                         acc.at[pl.ds(s * RPT + j * RCH, RCH)], semG[0])
    for j in range(nz):
        pltpu.make_async_copy(rows[0].at[pl.ds(0, RCH)],
                              acc.at[pl.ds(s * RPT + j * RCH, RCH)],
                              semG[0]).wait()

    plsc.subcore_barrier()
    base = s * (E // NS)
    nchunk = E // NS // KA
    ngrp = nchunk // NBUF

    def fire_idx(i, k):
        off = base + i * KA
        pltpu.async_copy(src_hbm.at[pl.ds(off, KA)], srcb[k], semI[k])
        pltpu.async_copy(dst_hbm.at[pl.ds(off, KA)], dstb[k], semI[k])

    def wait_idx(i, k):
        off = base + i * KA
        pltpu.make_async_copy(src_hbm.at[pl.ds(off, KA)], srcb[k],
                              semI[k]).wait()
        pltpu.make_async_copy(dst_hbm.at[pl.ds(off, KA)], dstb[k],
                              semI[k]).wait()

    def fire_gather(k):
        pltpu.async_copy(h2_hbm.at[srcb[k]], rows[k], semG[k])

    def wait_gather(k):
        pltpu.make_async_copy(h2_hbm.at[srcb[k]], rows[k], semG[k]).wait()

    for k in range(NBUF):
        fire_idx(k, k)
    wait_idx(0, 0)
    fire_gather(0)

    def grp(g, carry):
        for k in range(NBUF):
            kn = (k + 1) % NBUF

            def issue_next(gg):
                wait_idx(gg * NBUF + k + 1, kn)
                fire_gather(kn)

            if k < NBUF - 1:
                issue_next(g)
            else:
                @pl.when(g < ngrp - 1)
                def _():
                    issue_next(g)
            wait_gather(k)
            pltpu.sync_copy(rows[k], acc.at[dstb[k]], add=True)

            @pl.when(g < ngrp - 1)
            def _():
                fire_idx((g + 1) * NBUF + k, k)
        return carry

    lax.fori_loop(0, ngrp, grp, 0)
    plsc.subcore_barrier()

    def wb_ld(j, start):
        b = j % 2
        d = pltpu.async_copy if start else pltpu.make_async_copy
        r = d(acc.at[pl.ds(s * RPT + j * RCH, RCH)], rows[b], semG[b])
        if not start:
            r.wait()

    def wb_st(j, start):
        b = j % 2
        d = pltpu.async_copy if start else pltpu.make_async_copy
        r = d(rows[b], out_hbm.at[pl.ds(s * RPT + j * RCH, RCH)], semI[b])
        if not start:
            r.wait()

    nz2 = RPT // RCH
    wb_ld(0, True)
    wb_ld(1, True)
    for j in range(nz2):
        wb_ld(j, False)
        wb_st(j, True)
        if j + 2 < nz2:
            wb_st(j, False)
            wb_ld(j + 2, True)
        else:
            wb_st(j, False)


def _agg_call(src, dst, h2):
    fn = pl.kernel(
        _agg_body,
        out_type=jax.ShapeDtypeStruct((N, 32), jnp.bfloat16),
        mesh=_sc_mesh(),
        scratch_types=(
            [pltpu.VMEM((KA,), jnp.int32) for _ in range(2 * NBUF)]
            + [pltpu.VMEM((KA, 32), jnp.bfloat16) for _ in range(NBUF)]
            + [pltpu.SemaphoreType.DMA for _ in range(2 * NBUF)]
            + [pltpu.VMEM_SHARED((N, 32), jnp.bfloat16)]
        ),
        compiler_params=pltpu.CompilerParams(use_tc_tiling_on_sc=False),
    )
    return fn(src, dst, h2, jnp.zeros((RCH, 32), jnp.bfloat16))


def _sc_mesh():
    return plsc.VectorSubcoreMesh(core_axis_name="c", subcore_axis_name="s",
                                  num_cores=1)


def _deg_call(dst):
    fn = pl.kernel(
        _deg_body,
        out_type=jax.ShapeDtypeStruct((N,), jnp.float32),
        mesh=_sc_mesh(),
        scratch_types=[
            pltpu.VMEM((KD,), jnp.int32),
            pltpu.VMEM((KD,), jnp.float32),
            pltpu.VMEM((RCH,), jnp.float32),
            pltpu.VMEM_SHARED((N,), jnp.float32),
        ],
        compiler_params=pltpu.CompilerParams(use_tc_tiling_on_sc=False),
    )
    return fn(dst, jnp.zeros((RCH,), jnp.float32), jnp.ones((KD,), jnp.float32))


PG = 16
SBK = B * S // PG
RBK = B * R // PG


def _stats_body(sx_ref, rx_ref, ss_ref, rs_ref):
    i = pl.program_id(0)
    sx = sx_ref[...]
    rflat = rx_ref[...][:, :3]

    @pl.when(i == 0)
    def _():
        ss_ref[...] = jnp.zeros_like(ss_ref)
        rs_ref[...] = jnp.zeros_like(rs_ref)

    ss_ref[0, :] += jnp.sum(sx, axis=0)
    ss_ref[1, :] += jnp.sum(sx * sx, axis=0)
    rs_ref[0, :] += jnp.sum(rflat, axis=0)
    rs_ref[1, :] += jnp.sum(rflat * rflat, axis=0)


def _stats_call(s_x, r_x):
    return pl.pallas_call(
        _stats_body,
        grid=(PG,),
        in_specs=[
            pl.BlockSpec((SBK, 4), lambda i: (i, 0)),
            pl.BlockSpec((RBK, 8), lambda i: (i, 0)),
        ],
        out_specs=[
            pl.BlockSpec((2, 4), lambda i: (0, 0)),
            pl.BlockSpec((2, 3), lambda i: (0, 0)),
        ],
        out_shape=[
            jax.ShapeDtypeStruct((2, 4), jnp.float32),
            jax.ShapeDtypeStruct((2, 3), jnp.float32),
        ],
    )(s_x, r_x)


def _prologue_body(sx_ref, rx_ref, ss_ref, rs_ref, bnsw, bnsb, bnrw, bnrb,
                   lsw, lrw, s_out, r_out, ox_out):
    ns = float(B * S)
    mu = ss_ref[0, :] / ns
    var = ss_ref[1, :] / ns - mu * mu
    sn = (sx_ref[...] - mu) / jnp.sqrt(var + EPS) * bnsw[0] + bnsb[0]
    s_out[...] = _leaky(jnp.dot(sn, lsw[...].T, preferred_element_type=jnp.float32))

    rx = rx_ref[...]
    nr = float(B * R)
    mu_r = rs_ref[0, :] / nr
    var_r = rs_ref[1, :] / nr - mu_r * mu_r
    rn = (rx[:, :3] - mu_r) / jnp.sqrt(var_r + EPS) * bnrw[0] + bnrb[0]
    r_out[...] = _leaky(jnp.dot(rn, lrw[...].T, preferred_element_type=jnp.float32))

    ox_out[...] = rx.reshape(RBK // R, R, 8)[:, 0, 3:8]


def _prologue_call(s_x, r_x, bn_s_w, bn_s_b, bn_r_w, bn_r_b, lin_s_W, lin_r_W):
    sstats, rstats = _stats_call(s_x, r_x)
    return pl.pallas_call(
        _prologue_body,
        grid=(PG,),
        in_specs=[
            pl.BlockSpec((SBK, 4), lambda i: (i, 0)),
            pl.BlockSpec((RBK, 8), lambda i: (i, 0)),
            pl.BlockSpec((2, 4), lambda i: (0, 0)),
            pl.BlockSpec((2, 3), lambda i: (0, 0)),
            pl.BlockSpec((1, 4), lambda i: (0, 0)),
            pl.BlockSpec((1, 4), lambda i: (0, 0)),
            pl.BlockSpec((1, 3), lambda i: (0, 0)),
            pl.BlockSpec((1, 3), lambda i: (0, 0)),
            pl.BlockSpec((C, 4), lambda i: (0, 0)),
            pl.BlockSpec((C, 3), lambda i: (0, 0)),
        ],
        out_specs=[
            pl.BlockSpec((SBK, C), lambda i: (i, 0)),
            pl.BlockSpec((RBK, C), lambda i: (i, 0)),
            pl.BlockSpec((RBK // R, 5), lambda i: (i, 0)),
        ],
        out_shape=[
            jax.ShapeDtypeStruct((B * S, C), jnp.float32),
            jax.ShapeDtypeStruct((B * R, C), jnp.float32),
            jax.ShapeDtypeStruct((B, 5), jnp.float32),
        ],
    )(s_x, r_x, sstats, rstats, bn_s_w.reshape(1, 4), bn_s_b.reshape(1, 4),
      bn_r_w.reshape(1, 3), bn_r_b.reshape(1, 3), lin_s_W, lin_r_W)


def _split_h(h, h2_ref):
    rb = h.shape[0]
    h2_ref[...] = jnp.concatenate(
        [h, jnp.zeros((rb, 8), jnp.float32)], axis=1).astype(jnp.bfloat16)


def _dense0_body(x_ref, deg0_ref, w_ref, dinv_ref, h2_ref):
    deg = 1.0 + deg0_ref[:, 0]
    dinv = lax.rsqrt(deg)
    dinv_ref[:, 0] = dinv
    h = jnp.dot(x_ref[...], w_ref[...].T, preferred_element_type=jnp.float32)
    _split_h(h * dinv[:, None], h2_ref)


def _dense0_call(x0, deg0, W0):
    grid = (N // RB,)
    return pl.pallas_call(
        _dense0_body,
        grid=grid,
        in_specs=[
            pl.BlockSpec((RB, C), lambda i: (i, 0)),
            pl.BlockSpec((RB, 1), lambda i: (i, 0)),
            pl.BlockSpec((C, C), lambda i: (0, 0)),
        ],
        out_specs=[
            pl.BlockSpec((RB, 1), lambda i: (i, 0)),
            pl.BlockSpec((RB, 32), lambda i: (i, 0)),
        ],
        out_shape=[
            jax.ShapeDtypeStruct((N, 1), jnp.float32),
            jax.ShapeDtypeStruct((N, 32), jnp.bfloat16),
        ],
    )(x0, deg0.reshape(N, 1), W0)


def _dense_body(x_ref, agg0_ref, hp_ref, dinv_ref, b_ref, w_ref,
                xn_ref, h2_ref):
    dinv = dinv_ref[:, 0]
    agg = (agg0_ref[...].astype(jnp.float32)
           + hp_ref[...].astype(jnp.float32))[:, :C]
    xn = x_ref[...] + _leaky(dinv[:, None] * agg + b_ref[0])
    xn_ref[...] = xn
    h = jnp.dot(xn, w_ref[...].T, preferred_element_type=jnp.float32)
    _split_h(h * dinv[:, None], h2_ref)


def _dense_call(x, agg0, hp, dinv, b, Wnext):
    grid = (N // RB,)
    return pl.pallas_call(
        _dense_body,
        grid=grid,
        in_specs=[
            pl.BlockSpec((RB, C), lambda i: (i, 0)),
            pl.BlockSpec((RB, 32), lambda i: (i, 0)),
            pl.BlockSpec((RB, 32), lambda i: (i, 0)),
            pl.BlockSpec((RB, 1), lambda i: (i, 0)),
            pl.BlockSpec((1, C), lambda i: (0, 0)),
            pl.BlockSpec((C, C), lambda i: (0, 0)),
        ],
        out_specs=[
            pl.BlockSpec((RB, C), lambda i: (i, 0)),
            pl.BlockSpec((RB, 32), lambda i: (i, 0)),
        ],
        out_shape=[
            jax.ShapeDtypeStruct((N, C), jnp.float32),
            jax.ShapeDtypeStruct((N, 32), jnp.bfloat16),
        ],
    )(x, agg0, hp, dinv, b.reshape(1, C), Wnext)


def _dense_last_body(x_ref, agg0_ref, hp_ref, dinv_ref, b_ref, xn_ref):
    dinv = dinv_ref[:, 0]
    agg = (agg0_ref[...].astype(jnp.float32)
           + hp_ref[...].astype(jnp.float32))[:, :C]
    xn_ref[...] = x_ref[...] + _leaky(dinv[:, None] * agg + b_ref[0])


def _dense_last_call(x, agg0, hp, dinv, b):
    grid = (N // RB,)
    return pl.pallas_call(
        _dense_last_body,
        grid=grid,
        in_specs=[
            pl.BlockSpec((RB, C), lambda i: (i, 0)),
            pl.BlockSpec((RB, 32), lambda i: (i, 0)),
            pl.BlockSpec((RB, 32), lambda i: (i, 0)),
            pl.BlockSpec((RB, 1), lambda i: (i, 0)),
            pl.BlockSpec((1, C), lambda i: (0, 0)),
        ],
        out_specs=pl.BlockSpec((RB, C), lambda i: (i, 0)),
        out_shape=jax.ShapeDtypeStruct((N, C), jnp.float32),
    )(x, agg0, hp, dinv, b.reshape(1, C))


def _readout_body(x_ref, ox_ref, lrw, lrb, w1, b1, w2, b2, out_ref):
    xg = jnp.mean(x_ref[...], axis=2)
    logits = jnp.dot(xg, lrw[...].T, preferred_element_type=jnp.float32) + lrb[0]
    exl = jnp.exp(logits)
    p = exl / (jnp.sum(exl, axis=1, keepdims=True) + 1.0)
    o = _leaky(jnp.dot(ox_ref[...], w1[...].T, preferred_element_type=jnp.float32) + b1[0])
    o = jnp.dot(o, w2[...].T, preferred_element_type=jnp.float32) + b2[0]
    out_ref[...] = p * jnp.exp(o)


def _readout_call(x4, o_x, linr_W, linr_b, lino_W1, lino_b1, lino_W2, lino_b2):
    BB = 32
    grid = (B // BB,)
    return pl.pallas_call(
        _readout_body,
        grid=grid,
        in_specs=[
            pl.BlockSpec((BB, S + R, C), lambda i: (i, 0, 0)),
            pl.BlockSpec((BB, 5), lambda i: (i, 0)),
            pl.BlockSpec((7, S + R), lambda i: (0, 0)),
            pl.BlockSpec((1, 7), lambda i: (0, 0)),
            pl.BlockSpec((C, 5), lambda i: (0, 0)),
            pl.BlockSpec((1, C), lambda i: (0, 0)),
            pl.BlockSpec((7, C), lambda i: (0, 0)),
            pl.BlockSpec((1, 7), lambda i: (0, 0)),
        ],
        out_specs=pl.BlockSpec((BB, 7), lambda i: (i, 0)),
        out_shape=jax.ShapeDtypeStruct((B, 7), jnp.float32),
    )(x4.reshape(B, S + R, C), o_x, linr_W, linr_b.reshape(1, 7),
      lino_W1, lino_b1.reshape(1, C), lino_W2, lino_b2.reshape(1, 7))


def kernel(s_x, r_x, edge_index, bn_s_w, bn_s_b, bn_r_w, bn_r_b, lin_s_W,
           lin_r_W, conv_W, conv_b, linr_W, linr_b, lino_W1, lino_b1,
           lino_W2, lino_b2):
    src = edge_index[0]
    dst = edge_index[1]

    degp = _deg_call(dst)
    s_emb, r_emb, o_x = _prologue_call(
        s_x, r_x, bn_s_w, bn_s_b, bn_r_w, bn_r_b, lin_s_W, lin_r_W)
    x = jnp.concatenate(
        [s_emb.reshape(B, S, C), r_emb.reshape(B, R, C)], axis=1).reshape(N, C)

    dinv, h2 = _dense0_call(x, degp, conv_W[0])
    for l in range(NUM_LAYERS):
        agg = _agg_call(src, dst, h2)
        if l < NUM_LAYERS - 1:
            x, h2 = _dense_call(x, agg, h2, dinv, conv_b[l], conv_W[l + 1])
        else:
            x = _dense_last_call(x, agg, h2, dinv, conv_b[l])

    return _readout_call(x, o_x, linr_W, linr_b,
                         lino_W1, lino_b1, lino_W2, lino_b2)
